# Initial kernel scaffold; baseline (speedup 1.0000x reference)
import functools

import jax
import jax.numpy as jnp
from jax import lax
from jax.experimental import pallas as pl
from jax.experimental.pallas import tpu as pltpu
from jax.experimental.pallas import tpu_sc as plsc

_B = 16384
_D = 128
_L = 16
_MARGIN = 1.0
_NC, _NS = 2, 16
_NW = _NC * _NS
_PER_W = 2 * _B // _NW
_C = 128
_NCHUNK = _PER_W // _C


def _sc_partials(table, h_idx, t_idx, r_idx):
    mesh = plsc.VectorSubcoreMesh(core_axis_name="c", subcore_axis_name="s")

    @functools.partial(
        pl.kernel,
        out_type=jax.ShapeDtypeStruct((2 * _B, _L), jnp.float32),
        mesh=mesh,
        scratch_types=[
            pltpu.VMEM((_C,), jnp.int32),
            pltpu.VMEM((_C,), jnp.int32),
            pltpu.VMEM((_C,), jnp.int32),
            pltpu.VMEM((_C, _D), jnp.float32),
            pltpu.VMEM((_C, _D), jnp.float32),
            pltpu.VMEM((_C, _D), jnp.float32),
            pltpu.VMEM((_C, _L), jnp.float32),
            pltpu.SemaphoreType.DMA,
            pltpu.SemaphoreType.DMA,
            pltpu.SemaphoreType.DMA,
        ],
    )
    def k(table_hbm, hi_hbm, ti_hbm, ri_hbm, out_hbm,
          hi_v, ti_v, ri_v, h_v, t_v, r_v, o_v, s0, s1, s2):
        wid = lax.axis_index("s") * _NC + lax.axis_index("c")
        base = wid * _PER_W

        @pl.loop(0, _NCHUNK)
        def _(g):
            jb = base + g * _C
            pltpu.sync_copy(hi_hbm.at[pl.ds(jb, _C)], hi_v)
            pltpu.sync_copy(ti_hbm.at[pl.ds(jb, _C)], ti_v)
            pltpu.sync_copy(ri_hbm.at[pl.ds(jb, _C)], ri_v)
            ch = pltpu.async_copy(table_hbm.at[hi_v], h_v, s0)
            ct = pltpu.async_copy(table_hbm.at[ti_v], t_v, s1)
            cr = pltpu.async_copy(table_hbm.at[ri_v], r_v, s2)
            ch.wait()
            ct.wait()
            cr.wait()

            @pl.loop(0, _C)
            def _(i):
                acc = jnp.zeros((_L,), jnp.float32)
                for kk in range(_D // _L):
                    sl = pl.ds(kk * _L, _L)
                    d = h_v[i, sl] + r_v[i, sl] - t_v[i, sl]
                    acc = acc + d * d
                o_v[i, :] = acc

            pltpu.sync_copy(o_v, out_hbm.at[pl.ds(jb, _C)])

    return k(table, h_idx, t_idx, r_idx)


def _tc_loss(parts):
    def body(p_ref, o_ref):
        p = p_ref[...]
        s2 = jnp.sum(p, axis=2)
        dist = jnp.sqrt(s2)
        hinge = jnp.maximum(_MARGIN + dist[0] - dist[1], 0.0)
        o_ref[0, 0] = jnp.sum(hinge)

    return pl.pallas_call(
        body,
        out_shape=jax.ShapeDtypeStruct((1, 1), jnp.float32),
    )(parts)


def kernel(positive_triples, negative_triples, embeddings):
    pos = positive_triples.astype(jnp.int32)
    neg = negative_triples.astype(jnp.int32)
    idx = jnp.concatenate([pos, neg], axis=0)
    h_idx = idx[:, 0]
    t_idx = idx[:, 1]
    r_idx = idx[:, 2]
    parts = _sc_partials(embeddings, h_idx, t_idx, r_idx)
    loss = _tc_loss(parts.reshape(2, _B, _L))
    return loss[0, 0]


# speedup vs baseline: 1.1948x; 1.1948x over previous
import functools

import jax
import jax.numpy as jnp
from jax import lax
from jax.experimental import pallas as pl
from jax.experimental.pallas import tpu as pltpu
from jax.experimental.pallas import tpu_sc as plsc

_B = 16384
_D = 128
_L = 16
_MARGIN = 1.0
_NC, _NS = 2, 16
_NW = _NC * _NS
_PER_W = 2 * _B // _NW
_C = 128
_NCHUNK = _PER_W // _C


def _sc_partials(table, h_idx, t_idx, r_idx):
    mesh = plsc.VectorSubcoreMesh(core_axis_name="c", subcore_axis_name="s")

    @functools.partial(
        pl.kernel,
        out_type=jax.ShapeDtypeStruct((2 * _B, _L), jnp.float32),
        mesh=mesh,
        scratch_types=[
            pltpu.VMEM((_C,), jnp.int32),
            pltpu.VMEM((_C,), jnp.int32),
            pltpu.VMEM((_C,), jnp.int32),
            pltpu.VMEM((_C, _D), jnp.float32),
            pltpu.VMEM((_C, _D), jnp.float32),
            pltpu.VMEM((_C, _D), jnp.float32),
            pltpu.VMEM((_C, _L), jnp.float32),
            pltpu.SemaphoreType.DMA,
            pltpu.SemaphoreType.DMA,
            pltpu.SemaphoreType.DMA,
        ],
    )
    def k(table_hbm, hi_hbm, ti_hbm, ri_hbm, out_hbm,
          hi_v, ti_v, ri_v, h_v, t_v, r_v, o_v, s0, s1, s2):
        wid = lax.axis_index("s") * _NC + lax.axis_index("c")
        base = wid * _PER_W

        @pl.loop(0, _NCHUNK)
        def _(g):
            jb = base + g * _C
            pltpu.sync_copy(hi_hbm.at[pl.ds(jb, _C)], hi_v)
            pltpu.sync_copy(ti_hbm.at[pl.ds(jb, _C)], ti_v)
            pltpu.sync_copy(ri_hbm.at[pl.ds(jb, _C)], ri_v)
            ch = pltpu.async_copy(table_hbm.at[hi_v], h_v, s0)
            ct = pltpu.async_copy(table_hbm.at[ti_v], t_v, s1)
            cr = pltpu.async_copy(table_hbm.at[ri_v], r_v, s2)
            ch.wait()
            ct.wait()
            cr.wait()

            @pl.loop(0, _C)
            def _(i):
                acc = jnp.zeros((_L,), jnp.float32)
                for kk in range(_D // _L):
                    sl = pl.ds(kk * _L, _L)
                    d = h_v[i, sl] + r_v[i, sl] - t_v[i, sl]
                    acc = acc + d * d
                o_v[i, :] = acc

            pltpu.sync_copy(o_v, out_hbm.at[pl.ds(jb, _C)])

    return k(table, h_idx, t_idx, r_idx)


def _tc_loss(parts):
    def body(p_ref, o_ref):
        p = p_ref[...]
        s2 = jnp.sum(p, axis=2)
        dist = jnp.sqrt(s2)
        hinge = jnp.maximum(_MARGIN + dist[0] - dist[1], 0.0)
        o_ref[...] = jnp.sum(hinge)[None, None]

    return pl.pallas_call(
        body,
        out_shape=jax.ShapeDtypeStruct((1, 1), jnp.float32),
    )(parts)


def kernel(positive_triples, negative_triples, embeddings):
    pos = positive_triples.astype(jnp.int32)
    neg = negative_triples.astype(jnp.int32)
    idx = jnp.concatenate([pos, neg], axis=0)
    h_idx = idx[:, 0]
    t_idx = idx[:, 1]
    r_idx = idx[:, 2]
    parts = _sc_partials(embeddings, h_idx, t_idx, r_idx)
    loss = _tc_loss(parts.reshape(2, _B, _L))
    return loss[0, 0]


# speedup vs baseline: 1.1964x; 1.0013x over previous
import dataclasses
import functools

import jax
import jax.numpy as jnp
from jax import lax
from jax.experimental import pallas as pl
from jax.experimental.pallas import tpu as pltpu
from jax.experimental.pallas import tpu_sc as plsc

_B = 16384
_D = 128
_L = 16
_MARGIN = 1.0
_NC, _NS = 2, 16
_NW = _NC * _NS
_PER_W = 2 * _B // _NW
_C = 64
_NCHUNK = _PER_W // _C


def _sc_partials(table, pos_idx, neg_idx):
    mesh = plsc.VectorSubcoreMesh(core_axis_name="c", subcore_axis_name="s")
    cp = pltpu.CompilerParams()
    if "needs_layout_passes" in pltpu.CompilerParams.__dataclass_fields__:
        cp = dataclasses.replace(cp, needs_layout_passes=False)

    @functools.partial(
        pl.kernel,
        compiler_params=cp,
        out_type=jax.ShapeDtypeStruct((2 * _B, _L), jnp.float32),
        mesh=mesh,
        scratch_types=[
            pltpu.VMEM((3 * _PER_W,), jnp.int32),
            pltpu.VMEM((_PER_W,), jnp.int32),
            pltpu.VMEM((_PER_W,), jnp.int32),
            pltpu.VMEM((_PER_W,), jnp.int32),
            pltpu.VMEM((2, _C, _D), jnp.float32),
            pltpu.VMEM((2, _C, _D), jnp.float32),
            pltpu.VMEM((2, _C, _D), jnp.float32),
            pltpu.VMEM((2, _C, _L), jnp.float32),
            pltpu.SemaphoreType.DMA,
            pltpu.SemaphoreType.DMA,
            pltpu.SemaphoreType.DMA,
            pltpu.SemaphoreType.DMA,
        ],
    )
    def k(table_hbm, pos_hbm, neg_hbm, out_hbm,
          idx2_v, hi_v, ti_v, ri_v, h_v, t_v, r_v, o_v, g0, g1, so0, so1):
        wid = lax.axis_index("s") * _NC + lax.axis_index("c")
        base = wid * _PER_W

        @pl.when(wid < _NS)
        def _():
            pltpu.sync_copy(pos_hbm.at[pl.ds(wid * 3 * _PER_W, 3 * _PER_W)],
                            idx2_v)

        @pl.when(wid >= _NS)
        def _():
            pltpu.sync_copy(
                neg_hbm.at[pl.ds((wid - _NS) * 3 * _PER_W, 3 * _PER_W)],
                idx2_v)

        lane3 = lax.iota(jnp.int32, _L) * 3

        @pl.loop(0, _PER_W // _L)
        def _(m):
            rows3 = lane3 + m * (3 * _L)
            for col, dst in ((0, hi_v), (1, ti_v), (2, ri_v)):
                dst[pl.ds(m * _L, _L)] = plsc.load_gather(idx2_v, [rows3 + col])

        gsem = (g0, g1)
        osem = (so0, so1)

        def fire(g):
            buf = g % 2
            sl = pl.ds(g * _C, _C)
            return (
                pltpu.async_copy(table_hbm.at[hi_v.at[sl]], h_v.at[buf], gsem[buf]),
                pltpu.async_copy(table_hbm.at[ti_v.at[sl]], t_v.at[buf], gsem[buf]),
                pltpu.async_copy(table_hbm.at[ri_v.at[sl]], r_v.at[buf], gsem[buf]),
            )

        copies = {0: fire(0)}
        ostores = {}
        for g in range(_NCHUNK):
            buf = g % 2
            if g + 1 < _NCHUNK:
                copies[g + 1] = fire(g + 1)
            for c in copies.pop(g):
                c.wait()
            if g >= 2:
                ostores.pop(g - 2).wait()

            hb, tb, rb, ob = h_v.at[buf], t_v.at[buf], r_v.at[buf], o_v.at[buf]

            @pl.loop(0, _C, step=2)
            def _(i):
                for ii in range(2):
                    acc = jnp.zeros((_L,), jnp.float32)
                    for kk in range(_D // _L):
                        sl = pl.ds(kk * _L, _L)
                        d = hb[i + ii, sl] + rb[i + ii, sl] - tb[i + ii, sl]
                        acc = acc + d * d
                    ob[i + ii, :] = acc

            ostores[g] = pltpu.async_copy(
                ob, out_hbm.at[pl.ds(base + g * _C, _C)], osem[buf])

        for g in sorted(ostores):
            ostores.pop(g).wait()

    return k(table, pos_idx, neg_idx)


def _tc_loss(parts):
    def body(p_ref, o_ref):
        p = p_ref[...]
        s2 = jnp.sum(p, axis=2)
        dist = jnp.sqrt(s2)
        hinge = jnp.maximum(_MARGIN + dist[0] - dist[1], 0.0)
        o_ref[...] = jnp.sum(hinge)[None, None]

    return pl.pallas_call(
        body,
        out_shape=jax.ShapeDtypeStruct((1, 1), jnp.float32),
    )(parts)


def kernel(positive_triples, negative_triples, embeddings):
    pos = positive_triples.astype(jnp.int32).reshape(-1)
    neg = negative_triples.astype(jnp.int32).reshape(-1)
    parts = _sc_partials(embeddings, pos, neg)
    loss = _tc_loss(parts.reshape(2, _B, _L))
    return loss[0, 0]
